# baseline (speedup 1.0000x reference)
import functools

import jax
import jax.numpy as jnp
from jax import lax
from jax.experimental import pallas as pl
from jax.experimental.pallas import tpu as pltpu
from jax.experimental.pallas import tpu_sc as plsc

NUM_CLASS = 100000
INPUT_SIZE = 256
WORD_EMB_DIM = 300
TOP_K = 5
B = 64

BN1 = 2048
BN2 = 2048
NEG_INF = float("-inf")
INT_BIG = 2**30

_CONTRACT_RHS1 = (((1,), (1,)), ((), ()))
_CONTRACT_STD = (((1,), (0,)), ((), ()))


def _fc1_topk_body(x_ref, lab_ref, w1_ref, b1_ref, out_ref, tk_ref, cv_ref, ci_ref):
    j = pl.program_id(0)
    nb = pl.num_programs(0)

    @pl.when(j == 0)
    def _init():
        cv_ref[...] = jnp.full((B, 128), NEG_INF, jnp.float32)
        ci_ref[...] = jnp.zeros((B, 128), jnp.int32)

    logits = lax.dot_general(
        x_ref[...], w1_ref[...], _CONTRACT_RHS1,
        preferred_element_type=jnp.float32,
    ) + b1_ref[...]
    out_ref[...] = logits

    cols = jax.lax.broadcasted_iota(jnp.int32, (B, BN1), 1) + j * BN1
    lab = lab_ref[...]
    valid = (cols < NUM_CLASS) & (cols != lab)
    masked = jnp.where(valid, logits, NEG_INF)

    G = 8
    neg = jnp.full((G, 128), NEG_INF, jnp.float32)
    zero = jnp.zeros((G, 128), jnp.int32)
    lane_iota = jax.lax.broadcasted_iota(jnp.int32, (G, 128), 1)
    pad_v = jnp.full((G, 128 - (TOP_K - 1)), NEG_INF, jnp.float32)
    pad_i = jnp.zeros((G, 128 - (TOP_K - 1)), jnp.int32)

    emit_i = []
    for rg in range(B // G):
        r0, r1 = rg * G, (rg + 1) * G
        t1, t2, t3, t4 = neg, neg, neg, neg
        i1, i2, i3, i4 = zero, zero, zero, zero
        for ch in range(BN1 // 128):
            v = masked[r0:r1, ch * 128:(ch + 1) * 128]
            c = lane_iota + (j * BN1 + ch * 128)
            b1 = v > t1
            b2 = v > t2
            b3 = v > t3
            b4 = v > t4
            t4 = jnp.where(b4, jnp.where(b3, t3, v), t4)
            i4 = jnp.where(b4, jnp.where(b3, i3, c), i4)
            t3 = jnp.where(b3, jnp.where(b2, t2, v), t3)
            i3 = jnp.where(b3, jnp.where(b2, i2, c), i3)
            t2 = jnp.where(b2, jnp.where(b1, t1, v), t2)
            i2 = jnp.where(b2, jnp.where(b1, i1, c), i2)
            t1 = jnp.where(b1, v, t1)
            i1 = jnp.where(b1, c, i1)

        vals = [cv_ref[r0:r1, :], t1, t2, t3, t4]
        idxs = [ci_ref[r0:r1, :], i1, i2, i3, i4]
        top_v = []
        top_i = []
        for _ in range(TOP_K - 1):
            m = vals[0]
            for v in vals[1:]:
                m = jnp.maximum(m, v)
            mx = jnp.max(m, axis=1, keepdims=True)
            sel = None
            for v, ix in zip(vals, idxs):
                cand = jnp.where(v == mx, ix, INT_BIG)
                sel = cand if sel is None else jnp.minimum(sel, cand)
            sel = jnp.min(sel, axis=1, keepdims=True)
            top_v.append(mx)
            top_i.append(sel)
            vals = [jnp.where((v == mx) & (ix == sel), NEG_INF, v)
                    for v, ix in zip(vals, idxs)]

        cv_ref[r0:r1, :] = jnp.concatenate(top_v + [pad_v], axis=1)
        ci_ref[r0:r1, :] = jnp.concatenate(top_i + [pad_i], axis=1)
        emit_i.append(top_i)

    @pl.when(j == nb - 1)
    def _emit():
        for rg, top_i in enumerate(emit_i):
            tk_ref[rg * G:(rg + 1) * G, :] = jnp.concatenate(
                [lab[rg * G:(rg + 1) * G, :]] + top_i, axis=1)


def _fc1_topk(x, lab2d, W1, b1_2d):
    nb = pl.cdiv(NUM_CLASS, BN1)
    return pl.pallas_call(
        _fc1_topk_body,
        grid=(nb,),
        in_specs=[
            pl.BlockSpec((B, INPUT_SIZE), lambda j: (0, 0)),
            pl.BlockSpec((B, 1), lambda j: (0, 0)),
            pl.BlockSpec((BN1, INPUT_SIZE), lambda j: (j, 0)),
            pl.BlockSpec((1, BN1), lambda j: (0, j)),
        ],
        out_specs=[
            pl.BlockSpec((B, BN1), lambda j: (0, j)),
            pl.BlockSpec((B, TOP_K), lambda j: (0, 0)),
        ],
        out_shape=[
            jax.ShapeDtypeStruct((B, NUM_CLASS), jnp.float32),
            jax.ShapeDtypeStruct((B, TOP_K), jnp.int32),
        ],
        scratch_shapes=[
            pltpu.VMEM((B, 128), jnp.float32),
            pltpu.VMEM((B, 128), jnp.int32),
        ],
        compiler_params=pltpu.CompilerParams(
            dimension_semantics=("arbitrary",),
        ),
    )(x, lab2d, W1, b1_2d)


_GATHER_WORKERS = 8
_ROWS_PER_WORKER = (B * TOP_K) // _GATHER_WORKERS


def _sc_gather(table, idx_flat):
    mesh = plsc.VectorSubcoreMesh(core_axis_name="c", subcore_axis_name="s")
    rpw = _ROWS_PER_WORKER
    chunk = 20

    @functools.partial(
        pl.kernel,
        mesh=mesh,
        out_type=jax.ShapeDtypeStruct((B * TOP_K, WORD_EMB_DIM), jnp.float32),
        scratch_types=[
            pltpu.VMEM((rpw + 8,), jnp.int32),
            pltpu.VMEM((rpw, WORD_EMB_DIM), jnp.float32),
            pltpu.SemaphoreType.DMA,
        ],
    )
    def gather_k(table_hbm, idx_hbm, out_hbm, idx_v, rows_v, sem):
        wid = lax.axis_index("s") * 2 + lax.axis_index("c")

        @pl.when(wid < _GATHER_WORKERS)
        def _():
            base = wid * rpw
            pltpu.sync_copy(idx_hbm.at[pl.ds(base, rpw)], idx_v.at[pl.ds(0, rpw)])
            for c0 in range(0, rpw, chunk):
                copies = []
                for i in range(c0, c0 + chunk):
                    vec = idx_v[pl.ds((i // 16) * 16, 16)]
                    copies.append(
                        pltpu.async_copy(table_hbm.at[vec[i % 16]], rows_v.at[i], sem))
                for cp in copies:
                    cp.wait()
            pltpu.sync_copy(rows_v, out_hbm.at[pl.ds(base, rpw)])

    return gather_k(table, idx_flat)


_GW = 16


def _tc_gather_body(idx_ref, *refs):
    j = pl.program_id(0)
    out_ref = refs[-1]
    cols = []
    for k in range(_GW):
        lane = idx_ref[j * _GW + k] % 128
        blk = refs[k][...]
        onehot = (
            jax.lax.broadcasted_iota(jnp.int32, (WORD_EMB_DIM, 128), 1) == lane)
        cols.append(
            jnp.sum(jnp.where(onehot, blk, 0.0), axis=1).reshape(
                1, 1, WORD_EMB_DIM))
    out_ref[...] = jnp.concatenate(cols, axis=0)


def _tc_gather(tableT, idx_flat):
    grid_spec = pltpu.PrefetchScalarGridSpec(
        num_scalar_prefetch=1,
        grid=(B * TOP_K // _GW,),
        in_specs=[
            pl.BlockSpec(
                (WORD_EMB_DIM, 128),
                lambda j, idx, k=k: (0, idx[j * _GW + k] // 128))
            for k in range(_GW)
        ],
        out_specs=pl.BlockSpec((_GW, 1, WORD_EMB_DIM), lambda j, idx: (j, 0, 0)),
    )
    out = pl.pallas_call(
        _tc_gather_body,
        grid_spec=grid_spec,
        out_shape=jax.ShapeDtypeStruct((B * TOP_K, 1, WORD_EMB_DIM), jnp.float32),
        compiler_params=pltpu.CompilerParams(
            dimension_semantics=("arbitrary",),
        ),
    )(idx_flat, *([tableT] * _GW))
    return out.reshape(B * TOP_K, WORD_EMB_DIM)


def _fc2_body(g_ref, wmt_ref, bm_ref, x5_ref, w2_ref, b2_ref, out_ref, ff_ref):
    j = pl.program_id(0)

    @pl.when(j == 0)
    def _ff():
        e_sel = lax.dot_general(
            g_ref[...], wmt_ref[...], _CONTRACT_STD,
            preferred_element_type=jnp.float32,
        ) + bm_ref[...]
        ff_ref[...] = x5_ref[...] + e_sel

    res = lax.dot_general(
        ff_ref[...], w2_ref[...], _CONTRACT_RHS1,
        preferred_element_type=jnp.float32,
    ) + b2_ref[...]
    for t in range(TOP_K):
        out_ref[t] = res[t * B:(t + 1) * B, :]


def _fc2(g, WmT, bm_2d, x5, W2, b2_2d):
    nb = pl.cdiv(NUM_CLASS, BN2)
    m = B * TOP_K
    return pl.pallas_call(
        _fc2_body,
        grid=(nb,),
        in_specs=[
            pl.BlockSpec((m, WORD_EMB_DIM), lambda j: (0, 0)),
            pl.BlockSpec((WORD_EMB_DIM, INPUT_SIZE), lambda j: (0, 0)),
            pl.BlockSpec((1, INPUT_SIZE), lambda j: (0, 0)),
            pl.BlockSpec((m, INPUT_SIZE), lambda j: (0, 0)),
            pl.BlockSpec((BN2, INPUT_SIZE), lambda j: (j, 0)),
            pl.BlockSpec((1, BN2), lambda j: (0, j)),
        ],
        out_specs=pl.BlockSpec((TOP_K, B, BN2), lambda j: (0, 0, j)),
        out_shape=jax.ShapeDtypeStruct((TOP_K, B, NUM_CLASS), jnp.float32),
        scratch_shapes=[pltpu.VMEM((m, INPUT_SIZE), jnp.float32)],
        compiler_params=pltpu.CompilerParams(
            dimension_semantics=("arbitrary",),
        ),
    )(g, WmT, bm_2d, x5, W2, b2_2d)


def kernel(x, label, W1, b1, Wm, bm, W2, b2, word_emb_tab):
    lab2d = label.reshape(B, 1).astype(jnp.int32)
    fc1, topk = _fc1_topk(x, lab2d, W1, b1.reshape(1, NUM_CLASS))
    idx_tmaj = topk.T.reshape(B * TOP_K)
    g = _tc_gather(word_emb_tab.T, idx_tmaj)
    x5 = jnp.broadcast_to(x[None, :, :], (TOP_K, B, INPUT_SIZE)).reshape(
        B * TOP_K, INPUT_SIZE)
    fc2 = _fc2(g, Wm.T, bm.reshape(1, INPUT_SIZE), x5, W2,
               b2.reshape(1, NUM_CLASS))
    return fc1, jnp.transpose(fc2, (1, 0, 2)), topk


# speedup vs baseline: 1.1080x; 1.1080x over previous
import functools

import jax
import jax.numpy as jnp
from jax import lax
from jax.experimental import pallas as pl
from jax.experimental.pallas import tpu as pltpu
from jax.experimental.pallas import tpu_sc as plsc

NUM_CLASS = 100000
INPUT_SIZE = 256
WORD_EMB_DIM = 300
TOP_K = 5
B = 64

BN1 = 2048
BN2 = 2048
NEG_INF = float("-inf")
INT_BIG = 2**30

_CONTRACT_RHS1 = (((1,), (1,)), ((), ()))
_CONTRACT_STD = (((1,), (0,)), ((), ()))


def _fc1_topk_body(x_ref, lab_ref, w1_ref, b1_ref, out_ref, tk_ref, cv_ref, ci_ref):
    j = pl.program_id(0)
    nb = pl.num_programs(0)

    @pl.when(j == 0)
    def _init():
        cv_ref[...] = jnp.full((B, 128), NEG_INF, jnp.float32)
        ci_ref[...] = jnp.zeros((B, 128), jnp.int32)

    logits = lax.dot_general(
        x_ref[...], w1_ref[...], _CONTRACT_RHS1,
        preferred_element_type=jnp.float32,
    ) + b1_ref[...]
    out_ref[...] = logits

    cols = jax.lax.broadcasted_iota(jnp.int32, (B, BN1), 1) + j * BN1
    lab = lab_ref[...]
    valid = (cols < NUM_CLASS) & (cols != lab)
    masked = jnp.where(valid, logits, NEG_INF)

    G = 8
    neg = jnp.full((G, 128), NEG_INF, jnp.float32)
    zero = jnp.zeros((G, 128), jnp.int32)
    lane_iota = jax.lax.broadcasted_iota(jnp.int32, (G, 128), 1)
    pad_v = jnp.full((G, 128 - (TOP_K - 1)), NEG_INF, jnp.float32)
    pad_i = jnp.zeros((G, 128 - (TOP_K - 1)), jnp.int32)

    emit_i = []
    for rg in range(B // G):
        r0, r1 = rg * G, (rg + 1) * G
        t1, t2, t3, t4 = neg, neg, neg, neg
        i1, i2, i3, i4 = zero, zero, zero, zero
        for ch in range(BN1 // 128):
            v = masked[r0:r1, ch * 128:(ch + 1) * 128]
            c = lane_iota + (j * BN1 + ch * 128)
            b1 = v > t1
            b2 = v > t2
            b3 = v > t3
            b4 = v > t4
            t4 = jnp.where(b4, jnp.where(b3, t3, v), t4)
            i4 = jnp.where(b4, jnp.where(b3, i3, c), i4)
            t3 = jnp.where(b3, jnp.where(b2, t2, v), t3)
            i3 = jnp.where(b3, jnp.where(b2, i2, c), i3)
            t2 = jnp.where(b2, jnp.where(b1, t1, v), t2)
            i2 = jnp.where(b2, jnp.where(b1, i1, c), i2)
            t1 = jnp.where(b1, v, t1)
            i1 = jnp.where(b1, c, i1)

        vals = [cv_ref[r0:r1, :], t1, t2, t3, t4]
        idxs = [ci_ref[r0:r1, :], i1, i2, i3, i4]
        top_v = []
        top_i = []
        for _ in range(TOP_K - 1):
            m = vals[0]
            for v in vals[1:]:
                m = jnp.maximum(m, v)
            mx = jnp.max(m, axis=1, keepdims=True)
            sel = None
            for v, ix in zip(vals, idxs):
                cand = jnp.where(v == mx, ix, INT_BIG)
                sel = cand if sel is None else jnp.minimum(sel, cand)
            sel = jnp.min(sel, axis=1, keepdims=True)
            top_v.append(mx)
            top_i.append(sel)
            vals = [jnp.where((v == mx) & (ix == sel), NEG_INF, v)
                    for v, ix in zip(vals, idxs)]

        cv_ref[r0:r1, :] = jnp.concatenate(top_v + [pad_v], axis=1)
        ci_ref[r0:r1, :] = jnp.concatenate(top_i + [pad_i], axis=1)
        emit_i.append(top_i)

    @pl.when(j == nb - 1)
    def _emit():
        for rg, top_i in enumerate(emit_i):
            tk_ref[rg * G:(rg + 1) * G, :] = jnp.concatenate(
                [lab[rg * G:(rg + 1) * G, :]] + top_i, axis=1)


def _fc1_topk(x, lab2d, W1, b1_2d):
    nb = pl.cdiv(NUM_CLASS, BN1)
    return pl.pallas_call(
        _fc1_topk_body,
        grid=(nb,),
        in_specs=[
            pl.BlockSpec((B, INPUT_SIZE), lambda j: (0, 0)),
            pl.BlockSpec((B, 1), lambda j: (0, 0)),
            pl.BlockSpec((BN1, INPUT_SIZE), lambda j: (j, 0)),
            pl.BlockSpec((1, BN1), lambda j: (0, j)),
        ],
        out_specs=[
            pl.BlockSpec((B, BN1), lambda j: (0, j)),
            pl.BlockSpec((B, TOP_K), lambda j: (0, 0)),
        ],
        out_shape=[
            jax.ShapeDtypeStruct((B, NUM_CLASS), jnp.float32),
            jax.ShapeDtypeStruct((B, TOP_K), jnp.int32),
        ],
        scratch_shapes=[
            pltpu.VMEM((B, 128), jnp.float32),
            pltpu.VMEM((B, 128), jnp.int32),
        ],
        compiler_params=pltpu.CompilerParams(
            dimension_semantics=("arbitrary",),
        ),
    )(x, lab2d, W1, b1_2d)


_GATHER_WORKERS = 8
_ROWS_PER_WORKER = (B * TOP_K) // _GATHER_WORKERS


def _sc_gather(table, idx_flat):
    mesh = plsc.VectorSubcoreMesh(core_axis_name="c", subcore_axis_name="s")
    rpw = _ROWS_PER_WORKER
    chunk = 20

    @functools.partial(
        pl.kernel,
        mesh=mesh,
        out_type=jax.ShapeDtypeStruct((B * TOP_K, WORD_EMB_DIM), jnp.float32),
        scratch_types=[
            pltpu.VMEM((rpw + 8,), jnp.int32),
            pltpu.VMEM((rpw, WORD_EMB_DIM), jnp.float32),
            pltpu.SemaphoreType.DMA,
        ],
    )
    def gather_k(table_hbm, idx_hbm, out_hbm, idx_v, rows_v, sem):
        wid = lax.axis_index("s") * 2 + lax.axis_index("c")

        @pl.when(wid < _GATHER_WORKERS)
        def _():
            base = wid * rpw
            pltpu.sync_copy(idx_hbm.at[pl.ds(base, rpw)], idx_v.at[pl.ds(0, rpw)])
            for c0 in range(0, rpw, chunk):
                copies = []
                for i in range(c0, c0 + chunk):
                    vec = idx_v[pl.ds((i // 16) * 16, 16)]
                    copies.append(
                        pltpu.async_copy(table_hbm.at[vec[i % 16]], rows_v.at[i], sem))
                for cp in copies:
                    cp.wait()
            pltpu.sync_copy(rows_v, out_hbm.at[pl.ds(base, rpw)])

    return gather_k(table, idx_flat)


_GW = 16


def _tc_gather_body(idx_ref, *refs):
    j = pl.program_id(0)
    out_ref = refs[-1]
    for k in range(_GW):
        lane = idx_ref[j * _GW + k] % 128
        blk = refs[k][...]
        onehot = (
            (jax.lax.broadcasted_iota(jnp.int32, (128, 8), 0) == lane)
            & (jax.lax.broadcasted_iota(jnp.int32, (128, 8), 1) == 0)
        ).astype(jnp.float32)
        res = lax.dot_general(
            blk, onehot, _CONTRACT_STD, preferred_element_type=jnp.float32)
        out_ref[0, :, k:k + 1] = res[:, 0:1]


def _tc_gather(tableT, idx_flat):
    grid_spec = pltpu.PrefetchScalarGridSpec(
        num_scalar_prefetch=1,
        grid=(B * TOP_K // _GW,),
        in_specs=[
            pl.BlockSpec(
                (WORD_EMB_DIM, 128),
                lambda j, idx, k=k: (0, idx[j * _GW + k] // 128))
            for k in range(_GW)
        ],
        out_specs=pl.BlockSpec((1, WORD_EMB_DIM, _GW), lambda j, idx: (j, 0, 0)),
    )
    out = pl.pallas_call(
        _tc_gather_body,
        grid_spec=grid_spec,
        out_shape=jax.ShapeDtypeStruct(
            (B * TOP_K // _GW, WORD_EMB_DIM, _GW), jnp.float32),
        compiler_params=pltpu.CompilerParams(
            dimension_semantics=("arbitrary",),
        ),
    )(idx_flat, *([tableT] * _GW))
    return out.transpose(1, 0, 2).reshape(WORD_EMB_DIM, B * TOP_K)


def _fc2_body(g_ref, wmt_ref, bm_ref, x5_ref, w2_ref, b2_ref, out_ref, ff_ref):
    j = pl.program_id(0)

    @pl.when(j == 0)
    def _ff():
        e_sel = lax.dot_general(
            g_ref[...], wmt_ref[...], (((0,), (0,)), ((), ())),
            preferred_element_type=jnp.float32,
        ) + bm_ref[...]
        ff_ref[...] = x5_ref[...] + e_sel

    res = lax.dot_general(
        ff_ref[...], w2_ref[...], _CONTRACT_RHS1,
        preferred_element_type=jnp.float32,
    ) + b2_ref[...]
    for t in range(TOP_K):
        out_ref[t] = res[t * B:(t + 1) * B, :]


def _fc2(g, WmT, bm_2d, x5, W2, b2_2d):
    nb = pl.cdiv(NUM_CLASS, BN2)
    m = B * TOP_K
    return pl.pallas_call(
        _fc2_body,
        grid=(nb,),
        in_specs=[
            pl.BlockSpec((WORD_EMB_DIM, m), lambda j: (0, 0)),
            pl.BlockSpec((WORD_EMB_DIM, INPUT_SIZE), lambda j: (0, 0)),
            pl.BlockSpec((1, INPUT_SIZE), lambda j: (0, 0)),
            pl.BlockSpec((m, INPUT_SIZE), lambda j: (0, 0)),
            pl.BlockSpec((BN2, INPUT_SIZE), lambda j: (j, 0)),
            pl.BlockSpec((1, BN2), lambda j: (0, j)),
        ],
        out_specs=pl.BlockSpec((TOP_K, B, BN2), lambda j: (0, 0, j)),
        out_shape=jax.ShapeDtypeStruct((TOP_K, B, NUM_CLASS), jnp.float32),
        scratch_shapes=[pltpu.VMEM((m, INPUT_SIZE), jnp.float32)],
        compiler_params=pltpu.CompilerParams(
            dimension_semantics=("arbitrary",),
        ),
    )(g, WmT, bm_2d, x5, W2, b2_2d)


def kernel(x, label, W1, b1, Wm, bm, W2, b2, word_emb_tab):
    lab2d = label.reshape(B, 1).astype(jnp.int32)
    fc1, topk = _fc1_topk(x, lab2d, W1, b1.reshape(1, NUM_CLASS))
    idx_tmaj = topk.T.reshape(B * TOP_K)
    g = _tc_gather(word_emb_tab.T, idx_tmaj)
    x5 = jnp.broadcast_to(x[None, :, :], (TOP_K, B, INPUT_SIZE)).reshape(
        B * TOP_K, INPUT_SIZE)
    fc2 = _fc2(g, Wm.T, bm.reshape(1, INPUT_SIZE), x5, W2,
               b2.reshape(1, NUM_CLASS))
    return fc1, jnp.transpose(fc2, (1, 0, 2)), topk


# speedup vs baseline: 1.2424x; 1.1213x over previous
import functools

import jax
import jax.numpy as jnp
from jax import lax
from jax.experimental import pallas as pl
from jax.experimental.pallas import tpu as pltpu
from jax.experimental.pallas import tpu_sc as plsc

NUM_CLASS = 100000
INPUT_SIZE = 256
WORD_EMB_DIM = 300
TOP_K = 5
B = 64

BN1 = 4096
BN2 = 2048
NEG_INF = float("-inf")
INT_BIG = 2**30

_CONTRACT_RHS1 = (((1,), (1,)), ((), ()))
_CONTRACT_STD = (((1,), (0,)), ((), ()))


def _fc1_topk_body(x_ref, lab_ref, w1_ref, b1_ref, out_ref, tk_ref, cv_ref, ci_ref):
    j = pl.program_id(0)
    nb = pl.num_programs(0)

    @pl.when(j == 0)
    def _init():
        cv_ref[...] = jnp.full((B, 128), NEG_INF, jnp.float32)
        ci_ref[...] = jnp.zeros((B, 128), jnp.int32)

    logits = lax.dot_general(
        x_ref[...], w1_ref[...], _CONTRACT_RHS1,
        preferred_element_type=jnp.float32,
    ) + b1_ref[...]
    out_ref[...] = logits

    cols = jax.lax.broadcasted_iota(jnp.int32, (B, BN1), 1) + j * BN1
    lab = lab_ref[...]
    valid = (cols < NUM_CLASS) & (cols != lab)
    masked = jnp.where(valid, logits, NEG_INF)

    G = 8
    neg = jnp.full((G, 128), NEG_INF, jnp.float32)
    zero = jnp.zeros((G, 128), jnp.int32)
    lane_iota = jax.lax.broadcasted_iota(jnp.int32, (G, 128), 1)
    pad_v = jnp.full((G, 128 - (TOP_K - 1)), NEG_INF, jnp.float32)
    pad_i = jnp.zeros((G, 128 - (TOP_K - 1)), jnp.int32)

    emit_i = []
    for rg in range(B // G):
        r0, r1 = rg * G, (rg + 1) * G
        t1, t2, t3, t4 = neg, neg, neg, neg
        i1, i2, i3, i4 = zero, zero, zero, zero
        for ch in range(BN1 // 128):
            v = masked[r0:r1, ch * 128:(ch + 1) * 128]
            c = lane_iota + (j * BN1 + ch * 128)
            b1 = v > t1
            b2 = v > t2
            b3 = v > t3
            b4 = v > t4
            t4 = jnp.where(b4, jnp.where(b3, t3, v), t4)
            i4 = jnp.where(b4, jnp.where(b3, i3, c), i4)
            t3 = jnp.where(b3, jnp.where(b2, t2, v), t3)
            i3 = jnp.where(b3, jnp.where(b2, i2, c), i3)
            t2 = jnp.where(b2, jnp.where(b1, t1, v), t2)
            i2 = jnp.where(b2, jnp.where(b1, i1, c), i2)
            t1 = jnp.where(b1, v, t1)
            i1 = jnp.where(b1, c, i1)

        vals = [cv_ref[r0:r1, :], t1, t2, t3, t4]
        idxs = [ci_ref[r0:r1, :], i1, i2, i3, i4]
        top_v = []
        top_i = []
        for _ in range(TOP_K - 1):
            m = vals[0]
            for v in vals[1:]:
                m = jnp.maximum(m, v)
            mx = jnp.max(m, axis=1, keepdims=True)
            sel = None
            for v, ix in zip(vals, idxs):
                cand = jnp.where(v == mx, ix, INT_BIG)
                sel = cand if sel is None else jnp.minimum(sel, cand)
            sel = jnp.min(sel, axis=1, keepdims=True)
            top_v.append(mx)
            top_i.append(sel)
            vals = [jnp.where((v == mx) & (ix == sel), NEG_INF, v)
                    for v, ix in zip(vals, idxs)]

        cv_ref[r0:r1, :] = jnp.concatenate(top_v + [pad_v], axis=1)
        ci_ref[r0:r1, :] = jnp.concatenate(top_i + [pad_i], axis=1)
        emit_i.append(top_i)

    @pl.when(j == nb - 1)
    def _emit():
        for rg, top_i in enumerate(emit_i):
            tk_ref[rg * G:(rg + 1) * G, :] = jnp.concatenate(
                [lab[rg * G:(rg + 1) * G, :]] + top_i, axis=1)


def _fc1_topk(x, lab2d, W1, b1_2d):
    nb = pl.cdiv(NUM_CLASS, BN1)
    return pl.pallas_call(
        _fc1_topk_body,
        grid=(nb,),
        in_specs=[
            pl.BlockSpec((B, INPUT_SIZE), lambda j: (0, 0)),
            pl.BlockSpec((B, 1), lambda j: (0, 0)),
            pl.BlockSpec((BN1, INPUT_SIZE), lambda j: (j, 0)),
            pl.BlockSpec((1, BN1), lambda j: (0, j)),
        ],
        out_specs=[
            pl.BlockSpec((B, BN1), lambda j: (0, j)),
            pl.BlockSpec((B, TOP_K), lambda j: (0, 0)),
        ],
        out_shape=[
            jax.ShapeDtypeStruct((B, NUM_CLASS), jnp.float32),
            jax.ShapeDtypeStruct((B, TOP_K), jnp.int32),
        ],
        scratch_shapes=[
            pltpu.VMEM((B, 128), jnp.float32),
            pltpu.VMEM((B, 128), jnp.int32),
        ],
        compiler_params=pltpu.CompilerParams(
            dimension_semantics=("arbitrary",),
        ),
    )(x, lab2d, W1, b1_2d)


_GATHER_WORKERS = 8
_ROWS_PER_WORKER = (B * TOP_K) // _GATHER_WORKERS


def _sc_gather(table, idx_flat):
    mesh = plsc.VectorSubcoreMesh(core_axis_name="c", subcore_axis_name="s")
    rpw = _ROWS_PER_WORKER
    chunk = 20

    @functools.partial(
        pl.kernel,
        mesh=mesh,
        out_type=jax.ShapeDtypeStruct((B * TOP_K, WORD_EMB_DIM), jnp.float32),
        scratch_types=[
            pltpu.VMEM((rpw + 8,), jnp.int32),
            pltpu.VMEM((rpw, WORD_EMB_DIM), jnp.float32),
            pltpu.SemaphoreType.DMA,
        ],
    )
    def gather_k(table_hbm, idx_hbm, out_hbm, idx_v, rows_v, sem):
        wid = lax.axis_index("s") * 2 + lax.axis_index("c")

        @pl.when(wid < _GATHER_WORKERS)
        def _():
            base = wid * rpw
            pltpu.sync_copy(idx_hbm.at[pl.ds(base, rpw)], idx_v.at[pl.ds(0, rpw)])
            for c0 in range(0, rpw, chunk):
                copies = []
                for i in range(c0, c0 + chunk):
                    vec = idx_v[pl.ds((i // 16) * 16, 16)]
                    copies.append(
                        pltpu.async_copy(table_hbm.at[vec[i % 16]], rows_v.at[i], sem))
                for cp in copies:
                    cp.wait()
            pltpu.sync_copy(rows_v, out_hbm.at[pl.ds(base, rpw)])

    return gather_k(table, idx_flat)


_GW = 16


def _tc_gather_body(idx_ref, *refs):
    j = pl.program_id(0)
    out_ref = refs[-1]
    for k in range(_GW):
        lane = idx_ref[j * _GW + k] % 128
        blk = refs[k][...]
        onehot = (
            (jax.lax.broadcasted_iota(jnp.int32, (128, 8), 0) == lane)
            & (jax.lax.broadcasted_iota(jnp.int32, (128, 8), 1) == 0)
        ).astype(jnp.float32)
        res = lax.dot_general(
            blk, onehot, _CONTRACT_STD, preferred_element_type=jnp.float32)
        out_ref[0, :, k:k + 1] = res[:, 0:1]


def _tc_gather(tableT, idx_flat):
    grid_spec = pltpu.PrefetchScalarGridSpec(
        num_scalar_prefetch=1,
        grid=(B * TOP_K // _GW,),
        in_specs=[
            pl.BlockSpec(
                (WORD_EMB_DIM, 128),
                lambda j, idx, k=k: (0, idx[j * _GW + k] // 128))
            for k in range(_GW)
        ],
        out_specs=pl.BlockSpec((1, WORD_EMB_DIM, _GW), lambda j, idx: (j, 0, 0)),
    )
    out = pl.pallas_call(
        _tc_gather_body,
        grid_spec=grid_spec,
        out_shape=jax.ShapeDtypeStruct(
            (B * TOP_K // _GW, WORD_EMB_DIM, _GW), jnp.float32),
        compiler_params=pltpu.CompilerParams(
            dimension_semantics=("arbitrary",),
        ),
    )(idx_flat, *([tableT] * _GW))
    return out.transpose(1, 0, 2).reshape(WORD_EMB_DIM, B * TOP_K)


def _fc2_body(g_ref, wmt_ref, bm_ref, x5_ref, w2_ref, b2_ref, out_ref, ff_ref):
    j = pl.program_id(0)

    @pl.when(j == 0)
    def _ff():
        e_sel = lax.dot_general(
            g_ref[...], wmt_ref[...], (((0,), (0,)), ((), ())),
            preferred_element_type=jnp.float32,
        ) + bm_ref[...]
        ff_ref[...] = x5_ref[...] + e_sel

    res = lax.dot_general(
        ff_ref[...], w2_ref[...], _CONTRACT_RHS1,
        preferred_element_type=jnp.float32,
    ) + b2_ref[...]
    for t in range(TOP_K):
        out_ref[t] = res[t * B:(t + 1) * B, :]


def _fc2(g, WmT, bm_2d, x5, W2, b2_2d):
    nb = pl.cdiv(NUM_CLASS, BN2)
    m = B * TOP_K
    return pl.pallas_call(
        _fc2_body,
        grid=(nb,),
        in_specs=[
            pl.BlockSpec((WORD_EMB_DIM, m), lambda j: (0, 0)),
            pl.BlockSpec((WORD_EMB_DIM, INPUT_SIZE), lambda j: (0, 0)),
            pl.BlockSpec((1, INPUT_SIZE), lambda j: (0, 0)),
            pl.BlockSpec((m, INPUT_SIZE), lambda j: (0, 0)),
            pl.BlockSpec((BN2, INPUT_SIZE), lambda j: (j, 0)),
            pl.BlockSpec((1, BN2), lambda j: (0, j)),
        ],
        out_specs=pl.BlockSpec((TOP_K, B, BN2), lambda j: (0, 0, j)),
        out_shape=jax.ShapeDtypeStruct((TOP_K, B, NUM_CLASS), jnp.float32),
        scratch_shapes=[pltpu.VMEM((m, INPUT_SIZE), jnp.float32)],
        compiler_params=pltpu.CompilerParams(
            dimension_semantics=("arbitrary",),
        ),
    )(g, WmT, bm_2d, x5, W2, b2_2d)


def kernel(x, label, W1, b1, Wm, bm, W2, b2, word_emb_tab):
    lab2d = label.reshape(B, 1).astype(jnp.int32)
    fc1, topk = _fc1_topk(x, lab2d, W1, b1.reshape(1, NUM_CLASS))
    idx_tmaj = topk.T.reshape(B * TOP_K)
    g = _tc_gather(word_emb_tab.T, idx_tmaj)
    x5 = jnp.broadcast_to(x[None, :, :], (TOP_K, B, INPUT_SIZE)).reshape(
        B * TOP_K, INPUT_SIZE)
    fc2 = _fc2(g, Wm.T, bm.reshape(1, INPUT_SIZE), x5, W2,
               b2.reshape(1, NUM_CLASS))
    return fc1, jnp.transpose(fc2, (1, 0, 2)), topk


# speedup vs baseline: 1.3073x; 1.0523x over previous
import functools

import jax
import jax.numpy as jnp
from jax import lax
from jax.experimental import pallas as pl
from jax.experimental.pallas import tpu as pltpu
from jax.experimental.pallas import tpu_sc as plsc

NUM_CLASS = 100000
INPUT_SIZE = 256
WORD_EMB_DIM = 300
TOP_K = 5
B = 64

BN1 = 4096
BN2 = 4096
NEG_INF = float("-inf")
INT_BIG = 2**30

_CONTRACT_RHS1 = (((1,), (1,)), ((), ()))
_CONTRACT_STD = (((1,), (0,)), ((), ()))


def _fc1_topk_body(x_ref, lab_ref, w1_ref, b1_ref, out_ref, tk_ref, cv_ref, ci_ref):
    j = pl.program_id(0)
    nb = pl.num_programs(0)

    @pl.when(j == 0)
    def _init():
        cv_ref[...] = jnp.full((B, 128), NEG_INF, jnp.float32)
        ci_ref[...] = jnp.zeros((B, 128), jnp.int32)

    logits = lax.dot_general(
        x_ref[...], w1_ref[...], _CONTRACT_RHS1,
        preferred_element_type=jnp.float32,
    ) + b1_ref[...]
    out_ref[...] = logits

    cols = jax.lax.broadcasted_iota(jnp.int32, (B, BN1), 1) + j * BN1
    lab = lab_ref[...]
    valid = (cols < NUM_CLASS) & (cols != lab)
    masked = jnp.where(valid, logits, NEG_INF)

    G = 8
    neg = jnp.full((G, 128), NEG_INF, jnp.float32)
    zero = jnp.zeros((G, 128), jnp.int32)
    lane_iota = jax.lax.broadcasted_iota(jnp.int32, (G, 128), 1)
    pad_v = jnp.full((G, 128 - (TOP_K - 1)), NEG_INF, jnp.float32)
    pad_i = jnp.zeros((G, 128 - (TOP_K - 1)), jnp.int32)

    emit_i = []
    for rg in range(B // G):
        r0, r1 = rg * G, (rg + 1) * G
        t1, t2, t3, t4 = neg, neg, neg, neg
        i1, i2, i3, i4 = zero, zero, zero, zero
        for ch in range(BN1 // 128):
            v = masked[r0:r1, ch * 128:(ch + 1) * 128]
            c = lane_iota + (j * BN1 + ch * 128)
            b1 = v > t1
            b2 = v > t2
            b3 = v > t3
            b4 = v > t4
            t4 = jnp.where(b4, jnp.where(b3, t3, v), t4)
            i4 = jnp.where(b4, jnp.where(b3, i3, c), i4)
            t3 = jnp.where(b3, jnp.where(b2, t2, v), t3)
            i3 = jnp.where(b3, jnp.where(b2, i2, c), i3)
            t2 = jnp.where(b2, jnp.where(b1, t1, v), t2)
            i2 = jnp.where(b2, jnp.where(b1, i1, c), i2)
            t1 = jnp.where(b1, v, t1)
            i1 = jnp.where(b1, c, i1)

        vals = [cv_ref[r0:r1, :], t1, t2, t3, t4]
        idxs = [ci_ref[r0:r1, :], i1, i2, i3, i4]
        top_v = []
        top_i = []
        for _ in range(TOP_K - 1):
            m = vals[0]
            for v in vals[1:]:
                m = jnp.maximum(m, v)
            mx = jnp.max(m, axis=1, keepdims=True)
            sel = None
            for v, ix in zip(vals, idxs):
                cand = jnp.where(v == mx, ix, INT_BIG)
                sel = cand if sel is None else jnp.minimum(sel, cand)
            sel = jnp.min(sel, axis=1, keepdims=True)
            top_v.append(mx)
            top_i.append(sel)
            vals = [jnp.where((v == mx) & (ix == sel), NEG_INF, v)
                    for v, ix in zip(vals, idxs)]

        cv_ref[r0:r1, :] = jnp.concatenate(top_v + [pad_v], axis=1)
        ci_ref[r0:r1, :] = jnp.concatenate(top_i + [pad_i], axis=1)
        emit_i.append(top_i)

    @pl.when(j == nb - 1)
    def _emit():
        for rg, top_i in enumerate(emit_i):
            tk_ref[rg * G:(rg + 1) * G, :] = jnp.concatenate(
                [lab[rg * G:(rg + 1) * G, :]] + top_i, axis=1)


def _fc1_topk(x, lab2d, W1, b1_2d):
    nb = pl.cdiv(NUM_CLASS, BN1)
    return pl.pallas_call(
        _fc1_topk_body,
        grid=(nb,),
        in_specs=[
            pl.BlockSpec((B, INPUT_SIZE), lambda j: (0, 0)),
            pl.BlockSpec((B, 1), lambda j: (0, 0)),
            pl.BlockSpec((BN1, INPUT_SIZE), lambda j: (j, 0)),
            pl.BlockSpec((1, BN1), lambda j: (0, j)),
        ],
        out_specs=[
            pl.BlockSpec((B, BN1), lambda j: (0, j)),
            pl.BlockSpec((B, TOP_K), lambda j: (0, 0)),
        ],
        out_shape=[
            jax.ShapeDtypeStruct((B, NUM_CLASS), jnp.float32),
            jax.ShapeDtypeStruct((B, TOP_K), jnp.int32),
        ],
        scratch_shapes=[
            pltpu.VMEM((B, 128), jnp.float32),
            pltpu.VMEM((B, 128), jnp.int32),
        ],
        compiler_params=pltpu.CompilerParams(
            dimension_semantics=("arbitrary",),
        ),
    )(x, lab2d, W1, b1_2d)


_GATHER_WORKERS = 8
_ROWS_PER_WORKER = (B * TOP_K) // _GATHER_WORKERS


def _sc_gather(table, idx_flat):
    mesh = plsc.VectorSubcoreMesh(core_axis_name="c", subcore_axis_name="s")
    rpw = _ROWS_PER_WORKER
    chunk = 20

    @functools.partial(
        pl.kernel,
        mesh=mesh,
        out_type=jax.ShapeDtypeStruct((B * TOP_K, WORD_EMB_DIM), jnp.float32),
        scratch_types=[
            pltpu.VMEM((rpw + 8,), jnp.int32),
            pltpu.VMEM((rpw, WORD_EMB_DIM), jnp.float32),
            pltpu.SemaphoreType.DMA,
        ],
    )
    def gather_k(table_hbm, idx_hbm, out_hbm, idx_v, rows_v, sem):
        wid = lax.axis_index("s") * 2 + lax.axis_index("c")

        @pl.when(wid < _GATHER_WORKERS)
        def _():
            base = wid * rpw
            pltpu.sync_copy(idx_hbm.at[pl.ds(base, rpw)], idx_v.at[pl.ds(0, rpw)])
            for c0 in range(0, rpw, chunk):
                copies = []
                for i in range(c0, c0 + chunk):
                    vec = idx_v[pl.ds((i // 16) * 16, 16)]
                    copies.append(
                        pltpu.async_copy(table_hbm.at[vec[i % 16]], rows_v.at[i], sem))
                for cp in copies:
                    cp.wait()
            pltpu.sync_copy(rows_v, out_hbm.at[pl.ds(base, rpw)])

    return gather_k(table, idx_flat)


_GW = 16


def _tc_gather_body(idx_ref, *refs):
    j = pl.program_id(0)
    out_ref = refs[-1]
    for k in range(_GW):
        lane = idx_ref[j * _GW + k] % 128
        blk = refs[k][...]
        onehot = (
            (jax.lax.broadcasted_iota(jnp.int32, (128, 8), 0) == lane)
            & (jax.lax.broadcasted_iota(jnp.int32, (128, 8), 1) == 0)
        ).astype(jnp.float32)
        res = lax.dot_general(
            blk, onehot, _CONTRACT_STD, preferred_element_type=jnp.float32)
        out_ref[0, :, k:k + 1] = res[:, 0:1]


def _tc_gather(tableT, idx_flat):
    grid_spec = pltpu.PrefetchScalarGridSpec(
        num_scalar_prefetch=1,
        grid=(B * TOP_K // _GW,),
        in_specs=[
            pl.BlockSpec(
                (WORD_EMB_DIM, 128),
                lambda j, idx, k=k: (0, idx[j * _GW + k] // 128))
            for k in range(_GW)
        ],
        out_specs=pl.BlockSpec((1, WORD_EMB_DIM, _GW), lambda j, idx: (j, 0, 0)),
    )
    out = pl.pallas_call(
        _tc_gather_body,
        grid_spec=grid_spec,
        out_shape=jax.ShapeDtypeStruct(
            (B * TOP_K // _GW, WORD_EMB_DIM, _GW), jnp.float32),
        compiler_params=pltpu.CompilerParams(
            dimension_semantics=("arbitrary",),
        ),
    )(idx_flat, *([tableT] * _GW))
    return out.transpose(1, 0, 2).reshape(WORD_EMB_DIM, B * TOP_K)


def _fc2_body(g_ref, wmt_ref, bm_ref, x5_ref, w2_ref, b2_ref, out_ref, ff_ref):
    j = pl.program_id(0)

    @pl.when(j == 0)
    def _ff():
        e_sel = lax.dot_general(
            g_ref[...], wmt_ref[...], (((0,), (0,)), ((), ())),
            preferred_element_type=jnp.float32,
        ) + bm_ref[...]
        ff_ref[...] = x5_ref[...] + e_sel

    res = lax.dot_general(
        ff_ref[...], w2_ref[...], _CONTRACT_RHS1,
        preferred_element_type=jnp.float32,
    ) + b2_ref[...]
    for t in range(TOP_K):
        out_ref[t] = res[t * B:(t + 1) * B, :]


def _fc2(g, WmT, bm_2d, x5, W2, b2_2d):
    nb = pl.cdiv(NUM_CLASS, BN2)
    m = B * TOP_K
    return pl.pallas_call(
        _fc2_body,
        grid=(nb,),
        in_specs=[
            pl.BlockSpec((WORD_EMB_DIM, m), lambda j: (0, 0)),
            pl.BlockSpec((WORD_EMB_DIM, INPUT_SIZE), lambda j: (0, 0)),
            pl.BlockSpec((1, INPUT_SIZE), lambda j: (0, 0)),
            pl.BlockSpec((m, INPUT_SIZE), lambda j: (0, 0)),
            pl.BlockSpec((BN2, INPUT_SIZE), lambda j: (j, 0)),
            pl.BlockSpec((1, BN2), lambda j: (0, j)),
        ],
        out_specs=pl.BlockSpec((TOP_K, B, BN2), lambda j: (0, 0, j)),
        out_shape=jax.ShapeDtypeStruct((TOP_K, B, NUM_CLASS), jnp.float32),
        scratch_shapes=[pltpu.VMEM((m, INPUT_SIZE), jnp.float32)],
        compiler_params=pltpu.CompilerParams(
            dimension_semantics=("arbitrary",),
        ),
    )(g, WmT, bm_2d, x5, W2, b2_2d)


def kernel(x, label, W1, b1, Wm, bm, W2, b2, word_emb_tab):
    lab2d = label.reshape(B, 1).astype(jnp.int32)
    fc1, topk = _fc1_topk(x, lab2d, W1, b1.reshape(1, NUM_CLASS))
    idx_tmaj = topk.T.reshape(B * TOP_K)
    g = _tc_gather(word_emb_tab.T, idx_tmaj)
    x5 = jnp.broadcast_to(x[None, :, :], (TOP_K, B, INPUT_SIZE)).reshape(
        B * TOP_K, INPUT_SIZE)
    fc2 = _fc2(g, Wm.T, bm.reshape(1, INPUT_SIZE), x5, W2,
               b2.reshape(1, NUM_CLASS))
    return fc1, jnp.transpose(fc2, (1, 0, 2)), topk


# speedup vs baseline: 1.3928x; 1.0654x over previous
import functools

import jax
import jax.numpy as jnp
from jax import lax
from jax.experimental import pallas as pl
from jax.experimental.pallas import tpu as pltpu
from jax.experimental.pallas import tpu_sc as plsc

NUM_CLASS = 100000
INPUT_SIZE = 256
WORD_EMB_DIM = 300
TOP_K = 5
B = 64

BN1 = 8192
BN2 = 4096
NEG_INF = float("-inf")
INT_BIG = 2**30

_CONTRACT_RHS1 = (((1,), (1,)), ((), ()))
_CONTRACT_STD = (((1,), (0,)), ((), ()))


def _fc1_topk_body(x_ref, lab_ref, w1_ref, b1_ref, out_ref, tk_ref, cv_ref, ci_ref):
    j = pl.program_id(0)
    nb = pl.num_programs(0)

    @pl.when(j == 0)
    def _init():
        cv_ref[...] = jnp.full((B, 128), NEG_INF, jnp.float32)
        ci_ref[...] = jnp.zeros((B, 128), jnp.int32)

    logits = lax.dot_general(
        x_ref[...], w1_ref[...], _CONTRACT_RHS1,
        preferred_element_type=jnp.float32,
    ) + b1_ref[...]
    out_ref[...] = logits

    cols = jax.lax.broadcasted_iota(jnp.int32, (B, BN1), 1) + j * BN1
    lab = lab_ref[...]
    valid = (cols < NUM_CLASS) & (cols != lab)
    masked = jnp.where(valid, logits, NEG_INF)

    G = 8
    neg = jnp.full((G, 128), NEG_INF, jnp.float32)
    zero = jnp.zeros((G, 128), jnp.int32)
    lane_iota = jax.lax.broadcasted_iota(jnp.int32, (G, 128), 1)
    pad_v = jnp.full((G, 128 - (TOP_K - 1)), NEG_INF, jnp.float32)
    pad_i = jnp.zeros((G, 128 - (TOP_K - 1)), jnp.int32)

    emit_i = []
    for rg in range(B // G):
        r0, r1 = rg * G, (rg + 1) * G
        t1, t2, t3, t4 = neg, neg, neg, neg
        i1, i2, i3, i4 = zero, zero, zero, zero
        for ch in range(BN1 // 128):
            v = masked[r0:r1, ch * 128:(ch + 1) * 128]
            c = lane_iota + (j * BN1 + ch * 128)
            b1 = v > t1
            b2 = v > t2
            b3 = v > t3
            b4 = v > t4
            t4 = jnp.where(b4, jnp.where(b3, t3, v), t4)
            i4 = jnp.where(b4, jnp.where(b3, i3, c), i4)
            t3 = jnp.where(b3, jnp.where(b2, t2, v), t3)
            i3 = jnp.where(b3, jnp.where(b2, i2, c), i3)
            t2 = jnp.where(b2, jnp.where(b1, t1, v), t2)
            i2 = jnp.where(b2, jnp.where(b1, i1, c), i2)
            t1 = jnp.where(b1, v, t1)
            i1 = jnp.where(b1, c, i1)

        vals = [cv_ref[r0:r1, :], t1, t2, t3, t4]
        idxs = [ci_ref[r0:r1, :], i1, i2, i3, i4]
        top_v = []
        top_i = []
        for _ in range(TOP_K - 1):
            m = vals[0]
            for v in vals[1:]:
                m = jnp.maximum(m, v)
            mx = jnp.max(m, axis=1, keepdims=True)
            sel = None
            for v, ix in zip(vals, idxs):
                cand = jnp.where(v == mx, ix, INT_BIG)
                sel = cand if sel is None else jnp.minimum(sel, cand)
            sel = jnp.min(sel, axis=1, keepdims=True)
            top_v.append(mx)
            top_i.append(sel)
            vals = [jnp.where((v == mx) & (ix == sel), NEG_INF, v)
                    for v, ix in zip(vals, idxs)]

        cv_ref[r0:r1, :] = jnp.concatenate(top_v + [pad_v], axis=1)
        ci_ref[r0:r1, :] = jnp.concatenate(top_i + [pad_i], axis=1)
        emit_i.append(top_i)

    @pl.when(j == nb - 1)
    def _emit():
        for rg, top_i in enumerate(emit_i):
            tk_ref[rg * G:(rg + 1) * G, :] = jnp.concatenate(
                [lab[rg * G:(rg + 1) * G, :]] + top_i, axis=1)


def _fc1_topk(x, lab2d, W1, b1_2d):
    nb = pl.cdiv(NUM_CLASS, BN1)
    return pl.pallas_call(
        _fc1_topk_body,
        grid=(nb,),
        in_specs=[
            pl.BlockSpec((B, INPUT_SIZE), lambda j: (0, 0)),
            pl.BlockSpec((B, 1), lambda j: (0, 0)),
            pl.BlockSpec((BN1, INPUT_SIZE), lambda j: (j, 0)),
            pl.BlockSpec((1, BN1), lambda j: (0, j)),
        ],
        out_specs=[
            pl.BlockSpec((B, BN1), lambda j: (0, j)),
            pl.BlockSpec((B, TOP_K), lambda j: (0, 0)),
        ],
        out_shape=[
            jax.ShapeDtypeStruct((B, NUM_CLASS), jnp.float32),
            jax.ShapeDtypeStruct((B, TOP_K), jnp.int32),
        ],
        scratch_shapes=[
            pltpu.VMEM((B, 128), jnp.float32),
            pltpu.VMEM((B, 128), jnp.int32),
        ],
        compiler_params=pltpu.CompilerParams(
            dimension_semantics=("arbitrary",),
        ),
    )(x, lab2d, W1, b1_2d)


_GATHER_WORKERS = 8
_ROWS_PER_WORKER = (B * TOP_K) // _GATHER_WORKERS


def _sc_gather(table, idx_flat):
    mesh = plsc.VectorSubcoreMesh(core_axis_name="c", subcore_axis_name="s")
    rpw = _ROWS_PER_WORKER
    chunk = 20

    @functools.partial(
        pl.kernel,
        mesh=mesh,
        out_type=jax.ShapeDtypeStruct((B * TOP_K, WORD_EMB_DIM), jnp.float32),
        scratch_types=[
            pltpu.VMEM((rpw + 8,), jnp.int32),
            pltpu.VMEM((rpw, WORD_EMB_DIM), jnp.float32),
            pltpu.SemaphoreType.DMA,
        ],
    )
    def gather_k(table_hbm, idx_hbm, out_hbm, idx_v, rows_v, sem):
        wid = lax.axis_index("s") * 2 + lax.axis_index("c")

        @pl.when(wid < _GATHER_WORKERS)
        def _():
            base = wid * rpw
            pltpu.sync_copy(idx_hbm.at[pl.ds(base, rpw)], idx_v.at[pl.ds(0, rpw)])
            for c0 in range(0, rpw, chunk):
                copies = []
                for i in range(c0, c0 + chunk):
                    vec = idx_v[pl.ds((i // 16) * 16, 16)]
                    copies.append(
                        pltpu.async_copy(table_hbm.at[vec[i % 16]], rows_v.at[i], sem))
                for cp in copies:
                    cp.wait()
            pltpu.sync_copy(rows_v, out_hbm.at[pl.ds(base, rpw)])

    return gather_k(table, idx_flat)


_GW = 16


def _tc_gather_body(idx_ref, *refs):
    j = pl.program_id(0)
    out_ref = refs[-1]
    for k in range(_GW):
        lane = idx_ref[j * _GW + k] % 128
        blk = refs[k][...]
        onehot = (
            (jax.lax.broadcasted_iota(jnp.int32, (128, 8), 0) == lane)
            & (jax.lax.broadcasted_iota(jnp.int32, (128, 8), 1) == 0)
        ).astype(jnp.float32)
        res = lax.dot_general(
            blk, onehot, _CONTRACT_STD, preferred_element_type=jnp.float32)
        out_ref[0, :, k:k + 1] = res[:, 0:1]


def _tc_gather(tableT, idx_flat):
    grid_spec = pltpu.PrefetchScalarGridSpec(
        num_scalar_prefetch=1,
        grid=(B * TOP_K // _GW,),
        in_specs=[
            pl.BlockSpec(
                (WORD_EMB_DIM, 128),
                lambda j, idx, k=k: (0, idx[j * _GW + k] // 128))
            for k in range(_GW)
        ],
        out_specs=pl.BlockSpec((1, WORD_EMB_DIM, _GW), lambda j, idx: (j, 0, 0)),
    )
    out = pl.pallas_call(
        _tc_gather_body,
        grid_spec=grid_spec,
        out_shape=jax.ShapeDtypeStruct(
            (B * TOP_K // _GW, WORD_EMB_DIM, _GW), jnp.float32),
        compiler_params=pltpu.CompilerParams(
            dimension_semantics=("arbitrary",),
        ),
    )(idx_flat, *([tableT] * _GW))
    return out.transpose(1, 0, 2).reshape(WORD_EMB_DIM, B * TOP_K)


def _fc2_body(g_ref, wmt_ref, bm_ref, x5_ref, w2_ref, b2_ref, out_ref, ff_ref):
    j = pl.program_id(0)

    @pl.when(j == 0)
    def _ff():
        e_sel = lax.dot_general(
            g_ref[...], wmt_ref[...], (((0,), (0,)), ((), ())),
            preferred_element_type=jnp.float32,
        ) + bm_ref[...]
        ff_ref[...] = x5_ref[...] + e_sel

    res = lax.dot_general(
        ff_ref[...], w2_ref[...], _CONTRACT_RHS1,
        preferred_element_type=jnp.float32,
    ) + b2_ref[...]
    for t in range(TOP_K):
        out_ref[t] = res[t * B:(t + 1) * B, :]


def _fc2(g, WmT, bm_2d, x5, W2, b2_2d):
    nb = pl.cdiv(NUM_CLASS, BN2)
    m = B * TOP_K
    return pl.pallas_call(
        _fc2_body,
        grid=(nb,),
        in_specs=[
            pl.BlockSpec((WORD_EMB_DIM, m), lambda j: (0, 0)),
            pl.BlockSpec((WORD_EMB_DIM, INPUT_SIZE), lambda j: (0, 0)),
            pl.BlockSpec((1, INPUT_SIZE), lambda j: (0, 0)),
            pl.BlockSpec((m, INPUT_SIZE), lambda j: (0, 0)),
            pl.BlockSpec((BN2, INPUT_SIZE), lambda j: (j, 0)),
            pl.BlockSpec((1, BN2), lambda j: (0, j)),
        ],
        out_specs=pl.BlockSpec((TOP_K, B, BN2), lambda j: (0, 0, j)),
        out_shape=jax.ShapeDtypeStruct((TOP_K, B, NUM_CLASS), jnp.float32),
        scratch_shapes=[pltpu.VMEM((m, INPUT_SIZE), jnp.float32)],
        compiler_params=pltpu.CompilerParams(
            dimension_semantics=("arbitrary",),
        ),
    )(g, WmT, bm_2d, x5, W2, b2_2d)


def kernel(x, label, W1, b1, Wm, bm, W2, b2, word_emb_tab):
    lab2d = label.reshape(B, 1).astype(jnp.int32)
    fc1, topk = _fc1_topk(x, lab2d, W1, b1.reshape(1, NUM_CLASS))
    idx_tmaj = topk.T.reshape(B * TOP_K)
    g = _tc_gather(word_emb_tab.T, idx_tmaj)
    x5 = jnp.broadcast_to(x[None, :, :], (TOP_K, B, INPUT_SIZE)).reshape(
        B * TOP_K, INPUT_SIZE)
    fc2 = _fc2(g, Wm.T, bm.reshape(1, INPUT_SIZE), x5, W2,
               b2.reshape(1, NUM_CLASS))
    return fc1, jnp.transpose(fc2, (1, 0, 2)), topk


# speedup vs baseline: 1.4102x; 1.0125x over previous
import functools

import jax
import jax.numpy as jnp
from jax import lax
from jax.experimental import pallas as pl
from jax.experimental.pallas import tpu as pltpu
from jax.experimental.pallas import tpu_sc as plsc

NUM_CLASS = 100000
INPUT_SIZE = 256
WORD_EMB_DIM = 300
TOP_K = 5
B = 64

BN1 = 8192
BN2 = 8192
NEG_INF = float("-inf")
INT_BIG = 2**30

_CONTRACT_RHS1 = (((1,), (1,)), ((), ()))
_CONTRACT_STD = (((1,), (0,)), ((), ()))


def _fc1_topk_body(x_ref, lab_ref, w1_ref, b1_ref, out_ref, tk_ref, cv_ref, ci_ref):
    j = pl.program_id(0)
    nb = pl.num_programs(0)

    @pl.when(j == 0)
    def _init():
        cv_ref[...] = jnp.full((B, 128), NEG_INF, jnp.float32)
        ci_ref[...] = jnp.zeros((B, 128), jnp.int32)

    logits = lax.dot_general(
        x_ref[...], w1_ref[...], _CONTRACT_RHS1,
        preferred_element_type=jnp.float32,
    ) + b1_ref[...]
    out_ref[...] = logits

    cols = jax.lax.broadcasted_iota(jnp.int32, (B, BN1), 1) + j * BN1
    lab = lab_ref[...]
    valid = (cols < NUM_CLASS) & (cols != lab)
    masked = jnp.where(valid, logits, NEG_INF)

    G = 8
    neg = jnp.full((G, 128), NEG_INF, jnp.float32)
    zero = jnp.zeros((G, 128), jnp.int32)
    lane_iota = jax.lax.broadcasted_iota(jnp.int32, (G, 128), 1)
    pad_v = jnp.full((G, 128 - (TOP_K - 1)), NEG_INF, jnp.float32)
    pad_i = jnp.zeros((G, 128 - (TOP_K - 1)), jnp.int32)

    emit_i = []
    for rg in range(B // G):
        r0, r1 = rg * G, (rg + 1) * G
        t1, t2, t3, t4 = neg, neg, neg, neg
        i1, i2, i3, i4 = zero, zero, zero, zero
        for ch in range(BN1 // 128):
            v = masked[r0:r1, ch * 128:(ch + 1) * 128]
            c = lane_iota + (j * BN1 + ch * 128)
            b1 = v > t1
            b2 = v > t2
            b3 = v > t3
            b4 = v > t4
            t4 = jnp.where(b4, jnp.where(b3, t3, v), t4)
            i4 = jnp.where(b4, jnp.where(b3, i3, c), i4)
            t3 = jnp.where(b3, jnp.where(b2, t2, v), t3)
            i3 = jnp.where(b3, jnp.where(b2, i2, c), i3)
            t2 = jnp.where(b2, jnp.where(b1, t1, v), t2)
            i2 = jnp.where(b2, jnp.where(b1, i1, c), i2)
            t1 = jnp.where(b1, v, t1)
            i1 = jnp.where(b1, c, i1)

        vals = [cv_ref[r0:r1, :], t1, t2, t3, t4]
        idxs = [ci_ref[r0:r1, :], i1, i2, i3, i4]
        top_v = []
        top_i = []
        for _ in range(TOP_K - 1):
            m = vals[0]
            for v in vals[1:]:
                m = jnp.maximum(m, v)
            mx = jnp.max(m, axis=1, keepdims=True)
            sel = None
            for v, ix in zip(vals, idxs):
                cand = jnp.where(v == mx, ix, INT_BIG)
                sel = cand if sel is None else jnp.minimum(sel, cand)
            sel = jnp.min(sel, axis=1, keepdims=True)
            top_v.append(mx)
            top_i.append(sel)
            vals = [jnp.where((v == mx) & (ix == sel), NEG_INF, v)
                    for v, ix in zip(vals, idxs)]

        cv_ref[r0:r1, :] = jnp.concatenate(top_v + [pad_v], axis=1)
        ci_ref[r0:r1, :] = jnp.concatenate(top_i + [pad_i], axis=1)
        emit_i.append(top_i)

    @pl.when(j == nb - 1)
    def _emit():
        for rg, top_i in enumerate(emit_i):
            tk_ref[rg * G:(rg + 1) * G, :] = jnp.concatenate(
                [lab[rg * G:(rg + 1) * G, :]] + top_i, axis=1)


def _fc1_topk(x, lab2d, W1, b1_2d):
    nb = pl.cdiv(NUM_CLASS, BN1)
    return pl.pallas_call(
        _fc1_topk_body,
        grid=(nb,),
        in_specs=[
            pl.BlockSpec((B, INPUT_SIZE), lambda j: (0, 0)),
            pl.BlockSpec((B, 1), lambda j: (0, 0)),
            pl.BlockSpec((BN1, INPUT_SIZE), lambda j: (j, 0)),
            pl.BlockSpec((1, BN1), lambda j: (0, j)),
        ],
        out_specs=[
            pl.BlockSpec((B, BN1), lambda j: (0, j)),
            pl.BlockSpec((B, TOP_K), lambda j: (0, 0)),
        ],
        out_shape=[
            jax.ShapeDtypeStruct((B, NUM_CLASS), jnp.float32),
            jax.ShapeDtypeStruct((B, TOP_K), jnp.int32),
        ],
        scratch_shapes=[
            pltpu.VMEM((B, 128), jnp.float32),
            pltpu.VMEM((B, 128), jnp.int32),
        ],
        compiler_params=pltpu.CompilerParams(
            dimension_semantics=("arbitrary",),
        ),
    )(x, lab2d, W1, b1_2d)


_GATHER_WORKERS = 8
_ROWS_PER_WORKER = (B * TOP_K) // _GATHER_WORKERS


def _sc_gather(table, idx_flat):
    mesh = plsc.VectorSubcoreMesh(core_axis_name="c", subcore_axis_name="s")
    rpw = _ROWS_PER_WORKER
    chunk = 20

    @functools.partial(
        pl.kernel,
        mesh=mesh,
        out_type=jax.ShapeDtypeStruct((B * TOP_K, WORD_EMB_DIM), jnp.float32),
        scratch_types=[
            pltpu.VMEM((rpw + 8,), jnp.int32),
            pltpu.VMEM((rpw, WORD_EMB_DIM), jnp.float32),
            pltpu.SemaphoreType.DMA,
        ],
    )
    def gather_k(table_hbm, idx_hbm, out_hbm, idx_v, rows_v, sem):
        wid = lax.axis_index("s") * 2 + lax.axis_index("c")

        @pl.when(wid < _GATHER_WORKERS)
        def _():
            base = wid * rpw
            pltpu.sync_copy(idx_hbm.at[pl.ds(base, rpw)], idx_v.at[pl.ds(0, rpw)])
            for c0 in range(0, rpw, chunk):
                copies = []
                for i in range(c0, c0 + chunk):
                    vec = idx_v[pl.ds((i // 16) * 16, 16)]
                    copies.append(
                        pltpu.async_copy(table_hbm.at[vec[i % 16]], rows_v.at[i], sem))
                for cp in copies:
                    cp.wait()
            pltpu.sync_copy(rows_v, out_hbm.at[pl.ds(base, rpw)])

    return gather_k(table, idx_flat)


_GW = 16


def _tc_gather_body(idx_ref, *refs):
    j = pl.program_id(0)
    out_ref = refs[-1]
    for k in range(_GW):
        lane = idx_ref[j * _GW + k] % 128
        blk = refs[k][...]
        onehot = (
            (jax.lax.broadcasted_iota(jnp.int32, (128, 8), 0) == lane)
            & (jax.lax.broadcasted_iota(jnp.int32, (128, 8), 1) == 0)
        ).astype(jnp.float32)
        res = lax.dot_general(
            blk, onehot, _CONTRACT_STD, preferred_element_type=jnp.float32)
        out_ref[0, :, k:k + 1] = res[:, 0:1]


def _tc_gather(tableT, idx_flat):
    grid_spec = pltpu.PrefetchScalarGridSpec(
        num_scalar_prefetch=1,
        grid=(B * TOP_K // _GW,),
        in_specs=[
            pl.BlockSpec(
                (WORD_EMB_DIM, 128),
                lambda j, idx, k=k: (0, idx[j * _GW + k] // 128))
            for k in range(_GW)
        ],
        out_specs=pl.BlockSpec((1, WORD_EMB_DIM, _GW), lambda j, idx: (j, 0, 0)),
    )
    out = pl.pallas_call(
        _tc_gather_body,
        grid_spec=grid_spec,
        out_shape=jax.ShapeDtypeStruct(
            (B * TOP_K // _GW, WORD_EMB_DIM, _GW), jnp.float32),
        compiler_params=pltpu.CompilerParams(
            dimension_semantics=("arbitrary",),
        ),
    )(idx_flat, *([tableT] * _GW))
    return out.transpose(1, 0, 2).reshape(WORD_EMB_DIM, B * TOP_K)


def _fc2_body(g_ref, wmt_ref, bm_ref, x5_ref, w2_ref, b2_ref, out_ref, ff_ref):
    j = pl.program_id(0)

    @pl.when(j == 0)
    def _ff():
        e_sel = lax.dot_general(
            g_ref[...], wmt_ref[...], (((0,), (0,)), ((), ())),
            preferred_element_type=jnp.float32,
        ) + bm_ref[...]
        ff_ref[...] = x5_ref[...] + e_sel

    res = lax.dot_general(
        ff_ref[...], w2_ref[...], _CONTRACT_RHS1,
        preferred_element_type=jnp.float32,
    ) + b2_ref[...]
    for t in range(TOP_K):
        out_ref[t] = res[t * B:(t + 1) * B, :]


def _fc2(g, WmT, bm_2d, x5, W2, b2_2d):
    nb = pl.cdiv(NUM_CLASS, BN2)
    m = B * TOP_K
    return pl.pallas_call(
        _fc2_body,
        grid=(nb,),
        in_specs=[
            pl.BlockSpec((WORD_EMB_DIM, m), lambda j: (0, 0)),
            pl.BlockSpec((WORD_EMB_DIM, INPUT_SIZE), lambda j: (0, 0)),
            pl.BlockSpec((1, INPUT_SIZE), lambda j: (0, 0)),
            pl.BlockSpec((m, INPUT_SIZE), lambda j: (0, 0)),
            pl.BlockSpec((BN2, INPUT_SIZE), lambda j: (j, 0)),
            pl.BlockSpec((1, BN2), lambda j: (0, j)),
        ],
        out_specs=pl.BlockSpec((TOP_K, B, BN2), lambda j: (0, 0, j)),
        out_shape=jax.ShapeDtypeStruct((TOP_K, B, NUM_CLASS), jnp.float32),
        scratch_shapes=[pltpu.VMEM((m, INPUT_SIZE), jnp.float32)],
        compiler_params=pltpu.CompilerParams(
            dimension_semantics=("arbitrary",),
        ),
    )(g, WmT, bm_2d, x5, W2, b2_2d)


def kernel(x, label, W1, b1, Wm, bm, W2, b2, word_emb_tab):
    lab2d = label.reshape(B, 1).astype(jnp.int32)
    fc1, topk = _fc1_topk(x, lab2d, W1, b1.reshape(1, NUM_CLASS))
    idx_tmaj = topk.T.reshape(B * TOP_K)
    g = _tc_gather(word_emb_tab.T, idx_tmaj)
    x5 = jnp.broadcast_to(x[None, :, :], (TOP_K, B, INPUT_SIZE)).reshape(
        B * TOP_K, INPUT_SIZE)
    fc2 = _fc2(g, Wm.T, bm.reshape(1, INPUT_SIZE), x5, W2,
               b2.reshape(1, NUM_CLASS))
    return fc1, jnp.transpose(fc2, (1, 0, 2)), topk
